# Initial kernel scaffold; baseline (speedup 1.0000x reference)
import functools

import jax
import jax.numpy as jnp
from jax import lax
from jax.experimental import pallas as pl
from jax.experimental.pallas import tpu as pltpu
from jax.experimental.pallas import tpu_sc as plsc

_NUM_ENTITIES = 1000000
_D = 32
_B = 16384 * 26

_NC = 2
_NS = 16
_NW = _NC * _NS
_BPW = _B // _NW

_CHUNK = 128
_GROUP = 8
_ROWS = _CHUNK * _GROUP
_NG = _BPW // _ROWS

assert _BPW % _ROWS == 0


def _gather_body(table_hbm, idx_hbm, out_hbm, idx_v, rows_v, sem):
    wid = lax.axis_index("s") * _NC + lax.axis_index("c")
    base = wid * _BPW
    pltpu.sync_copy(idx_hbm.at[pl.ds(base, _BPW)], idx_v)

    def body(g, carry):
        off = g * _ROWS
        copies = []
        for j in range(_GROUP):
            copies.append(pltpu.async_copy(
                table_hbm.at[idx_v.at[pl.ds(off + j * _CHUNK, _CHUNK)]],
                rows_v.at[pl.ds(j * _CHUNK, _CHUNK)],
                sem))
        for c in copies:
            c.wait()
        pltpu.sync_copy(rows_v, out_hbm.at[pl.ds(base + off, _ROWS)])
        return carry

    lax.fori_loop(0, _NG, body, 0)


_mesh = plsc.VectorSubcoreMesh(core_axis_name="c", subcore_axis_name="s")

_sc_gather = functools.partial(
    pl.kernel,
    out_type=jax.ShapeDtypeStruct((_B, _D), jnp.float32),
    mesh=_mesh,
    scratch_types=[
        pltpu.VMEM((_BPW,), jnp.int32),
        pltpu.VMEM((_ROWS, _D), jnp.float32),
        pltpu.SemaphoreType.DMA,
    ],
)(_gather_body)


def kernel(ents, table):
    idx = ents.reshape(-1).astype(jnp.int32)
    out = _sc_gather(table, idx)
    return out.reshape(ents.shape + (_D,))


# speedup vs baseline: 1.5585x; 1.5585x over previous
import functools

import jax
import jax.numpy as jnp
from jax import lax
from jax.experimental import pallas as pl
from jax.experimental.pallas import tpu as pltpu
from jax.experimental.pallas import tpu_sc as plsc

_NUM_ENTITIES = 1000000
_D = 32
_B = 16384 * 26

_NC = 2
_NS = 16
_NW = _NC * _NS
_BPW = _B // _NW

_CHUNK = 128
_GROUP = 8
_ROWS = _CHUNK * _GROUP
_NG = _BPW // _ROWS

assert _BPW % _ROWS == 0


def _gather_body(table_hbm, idx_hbm, out_hbm, idx_v, rows_v, sem):
    wid = lax.axis_index("s") * _NC + lax.axis_index("c")
    base = wid * _BPW
    pltpu.sync_copy(idx_hbm.at[pl.ds(base, _BPW)], idx_v)

    def body(g, carry):
        off = g * _ROWS
        copies = []
        for j in range(_GROUP):
            copies.append(pltpu.async_copy(
                table_hbm.at[idx_v.at[pl.ds(off + j * _CHUNK, _CHUNK)]],
                rows_v.at[pl.ds(j * _CHUNK, _CHUNK)],
                sem))
        for c in copies:
            c.wait()
        pltpu.sync_copy(rows_v, out_hbm.at[pl.ds(base + off, _ROWS)])
        return carry

    lax.fori_loop(0, _NG, body, 0)


_mesh = plsc.VectorSubcoreMesh(core_axis_name="c", subcore_axis_name="s")

_sc_gather = functools.partial(
    pl.kernel,
    out_type=jax.ShapeDtypeStruct((_B, _D), jnp.float32),
    mesh=_mesh,
    scratch_types=[
        pltpu.VMEM((_BPW,), jnp.int32),
        pltpu.VMEM((_ROWS, _D), jnp.float32),
        pltpu.SemaphoreType.DMA,
    ],
    compiler_params=pltpu.CompilerParams(use_tc_tiling_on_sc=False),
)(_gather_body)


def kernel(ents, table):
    idx = ents.reshape(-1).astype(jnp.int32)
    out = _sc_gather(table, idx)
    return out.reshape(ents.shape + (_D,))


# speedup vs baseline: 1.5694x; 1.0070x over previous
import functools

import jax
import jax.numpy as jnp
from jax import lax
from jax.experimental import pallas as pl
from jax.experimental.pallas import tpu as pltpu
from jax.experimental.pallas import tpu_sc as plsc

_NUM_ENTITIES = 1000000
_D = 32
_B = 16384 * 26

_NC = 2
_NS = 16
_NW = _NC * _NS
_BPW = _B // _NW

_CHUNK = 128
_GROUP = 4
_ROWS = _CHUNK * _GROUP
_NG = _BPW // _ROWS
_NBUF = 2

assert _BPW % _ROWS == 0 and (_NG - _NBUF) % _NBUF == 0


def _gather_body(table_hbm, idx_hbm, out_hbm, idx_v, rows_v, gsems, ssems):
    wid = lax.axis_index("s") * _NC + lax.axis_index("c")
    base = wid * _BPW
    pltpu.sync_copy(idx_hbm.at[pl.ds(base, _BPW)], idx_v)

    def fire_gather(g, b):
        copies = []
        for j in range(_GROUP):
            copies.append(pltpu.async_copy(
                table_hbm.at[idx_v.at[pl.ds(g * _ROWS + j * _CHUNK, _CHUNK)]],
                rows_v.at[b].at[pl.ds(j * _CHUNK, _CHUNK)],
                gsems.at[b]))
        return copies

    def wait_gather(g, b):
        for j in range(_GROUP):
            pltpu.make_async_copy(
                table_hbm.at[idx_v.at[pl.ds(g * _ROWS + j * _CHUNK, _CHUNK)]],
                rows_v.at[b].at[pl.ds(j * _CHUNK, _CHUNK)],
                gsems.at[b]).wait()

    def fire_store(g, b):
        return pltpu.async_copy(
            rows_v.at[b], out_hbm.at[pl.ds(base + g * _ROWS, _ROWS)],
            ssems.at[b])

    def wait_store(g, b):
        pltpu.make_async_copy(
            rows_v.at[b], out_hbm.at[pl.ds(base + g * _ROWS, _ROWS)],
            ssems.at[b]).wait()

    for b in range(_NBUF):
        fire_gather(b, b)

    @pl.loop(0, _NG - _NBUF, step=_NBUF)
    def _main(g0):
        for b in range(_NBUF):
            wait_gather(g0 + b, b)
            fire_store(g0 + b, b)
        for b in range(_NBUF):
            wait_store(g0 + b, b)
            fire_gather(g0 + _NBUF + b, b)

    for b in range(_NBUF):
        g = _NG - _NBUF + b
        wait_gather(g, b)
        fire_store(g, b)
    for b in range(_NBUF):
        wait_store(_NG - _NBUF + b, b)


_mesh = plsc.VectorSubcoreMesh(core_axis_name="c", subcore_axis_name="s")

_sc_gather = functools.partial(
    pl.kernel,
    out_type=jax.ShapeDtypeStruct((_B, _D), jnp.float32),
    mesh=_mesh,
    scratch_types=[
        pltpu.VMEM((_BPW,), jnp.int32),
        pltpu.VMEM((_NBUF, _ROWS, _D), jnp.float32),
        pltpu.SemaphoreType.DMA((_NBUF,)),
        pltpu.SemaphoreType.DMA((_NBUF,)),
    ],
    compiler_params=pltpu.CompilerParams(use_tc_tiling_on_sc=False),
)(_gather_body)


def kernel(ents, table):
    idx = ents.reshape(-1).astype(jnp.int32)
    out = _sc_gather(table, idx)
    return out.reshape(ents.shape + (_D,))
